# baseline (speedup 1.0000x reference)
import functools

import jax
import jax.numpy as jnp
import numpy as np
from jax import lax
from jax.experimental import pallas as pl
from jax.experimental.pallas import tpu as pltpu
from jax.experimental.pallas import tpu_sc as plsc

FROM_H = FROM_W = 224
TO_H = TO_W = 512
BATCH = 4
CHANNELS = 96
IMGS = BATCH * CHANNELS

SC_IMGS = 384
TC_IMGS = IMGS - SC_IMGS

NW = 32
NIMG_PER_W = SC_IMGS // NW
CHUNK = 4
SPI = (FROM_H // 7) // CHUNK
STEPS = NIMG_PER_W * SPI
IN_ROWS = CHUNK * 7
OUT_ROWS = CHUNK * 16
INW = IN_ROWS * FROM_W
NCHUNK16 = INW // 16
NBUF = 2

P7 = tuple(int(v) for v in (np.arange(7) * 16) // 7)


def _build_idx_table() -> np.ndarray:
    xmap = (np.arange(FROM_W) * TO_W) // FROM_W
    idx = np.empty((CHUNK, 7, FROM_W), np.int32)
    for rg in range(CHUNK):
        for j in range(7):
            idx[rg, j, :] = ((rg * 16 + P7[j]) << 9) | xmap
    return idx.reshape(-1)


def _build_s() -> np.ndarray:
    xmap = (np.arange(FROM_W) * TO_W) // FROM_W
    s = np.zeros((FROM_W, TO_W), np.float32)
    s[np.arange(FROM_W), xmap] = 1.0
    return s


_IDX_TABLE = _build_idx_table()
_S_MAT = _build_s()


HALF_ROWS = FROM_H // 2
HALF_OUT = TO_H // 2


def _sc_body(x_hbm, idx_hbm, out_hbm, idxv,
             in0, in1, out0, out1, isem0, isem1, osem0, osem1):
    inbufs = (in0, in1)
    outbufs = (out0, out1)
    isems = (isem0, isem1)
    osems = (osem0, osem1)

    wid = lax.axis_index("s") * 2 + lax.axis_index("c")
    img_base = wid * NIMG_PER_W

    pltpu.sync_copy(idx_hbm, idxv)

    zv = jnp.zeros((16,), jnp.float32)

    @pl.loop(0, OUT_ROWS)
    def _zero_r(r):
        for b in range(NBUF):
            @pl.loop(0, TO_W // 16, unroll=8)
            def _zero_c(c):
                outbufs[b][r, pl.ds(c * 16, 16)] = zv

    def in_slice(img, half):
        return x_hbm.at[img, pl.ds(half * HALF_ROWS, HALF_ROWS)]

    for half in range(2):
        pltpu.async_copy(in_slice(img_base, half), inbufs[half], isems[half])

    @pl.loop(0, NIMG_PER_W)
    def _image(li):
        img = img_base + li
        for half in range(2):
            pltpu.make_async_copy(
                in_slice(img, half), inbufs[half], isems[half]).wait()
            inb = inbufs[half]
            for sub in range(4):
                b = sub % 2
                outb = outbufs[b]
                r0 = half * HALF_OUT + sub * OUT_ROWS
                out_ref = out_hbm.at[img, pl.ds(r0, OUT_ROWS)]

                if half == 0 and sub < 2:
                    @pl.when(li > 0)
                    def _wait_out():
                        pltpu.make_async_copy(outb, out_ref, osems[b]).wait()
                else:
                    pltpu.make_async_copy(outb, out_ref, osems[b]).wait()

                row0 = sub * IN_ROWS

                @plsc.parallel_loop(0, NCHUNK16, 1, unroll=14)
                def _scatter(ci):
                    r = ci // (FROM_W // 16)
                    c = ci % (FROM_W // 16)
                    iv = idxv[pl.ds(ci * 16, 16)]
                    ivr = jax.lax.shift_right_logical(iv, 9)
                    ivc = jax.lax.bitwise_and(iv, 511)
                    dv = inb[row0 + r, pl.ds(c * 16, 16)]
                    plsc.store_scatter(outb, [ivr, ivc], dv)

                pltpu.async_copy(outb, out_ref, osems[b])

            @pl.when(li + 1 < NIMG_PER_W)
            def _next_in():
                pltpu.async_copy(
                    in_slice(img + 1, half), inbufs[half], isems[half])

    for b in range(2):
        pltpu.make_async_copy(
            outbufs[b], out_hbm.at[img_base, pl.ds(0, OUT_ROWS)],
            osems[b]).wait()


def _sc_dilute(x_img, idx):
    call = pl.kernel(
        _sc_body,
        out_type=jax.ShapeDtypeStruct((SC_IMGS, TO_H, TO_W), jnp.float32),
        mesh=plsc.VectorSubcoreMesh(core_axis_name="c", subcore_axis_name="s"),
        compiler_params=pltpu.CompilerParams(needs_layout_passes=False),
        scratch_types=(
            [pltpu.VMEM((INW,), jnp.int32)]
            + [pltpu.VMEM((HALF_ROWS, FROM_W), jnp.float32) for _ in range(2)]
            + [pltpu.VMEM((OUT_ROWS, TO_W), jnp.float32) for _ in range(2)]
            + [pltpu.SemaphoreType.DMA for _ in range(4)]
        ),
    )
    return call(x_img, idx)


def _tc_body(x_ref, s_ref, o_ref):
    img = x_ref[0]
    mm = jnp.dot(img, s_ref[...], preferred_element_type=jnp.float32)
    d = mm.reshape(FROM_H // 7, 7, TO_W)
    z = jnp.zeros((FROM_H // 7, 1, TO_W), jnp.float32)
    out = jnp.concatenate(
        [d[:, 0:1], z, d[:, 1:2], z, d[:, 2:3], z, d[:, 3:4], z, z,
         d[:, 4:5], z, d[:, 5:6], z, d[:, 6:7], z, z], axis=1)
    o_ref[0] = out.reshape(TO_H, TO_W)


def _tc_dilute(x_img, s):
    return pl.pallas_call(
        _tc_body,
        out_shape=jax.ShapeDtypeStruct((IMGS, TO_H, TO_W), jnp.float32),
        grid=(TC_IMGS,),
        in_specs=[
            pl.BlockSpec((1, FROM_H, FROM_W), lambda i: (SC_IMGS + i, 0, 0)),
            pl.BlockSpec((FROM_W, TO_W), lambda i: (0, 0)),
        ],
        out_specs=pl.BlockSpec((1, TO_H, TO_W), lambda i: (SC_IMGS + i, 0, 0)),
    )(x_img, s)


@jax.jit
def _dilute(x):
    x_img = x.reshape(IMGS, FROM_H, FROM_W)
    idx = jnp.asarray(_IDX_TABLE)
    s = jnp.asarray(_S_MAT)
    sc_out = _sc_dilute(x_img, idx)
    if TC_IMGS == 0:
        return sc_out
    tc_out = _tc_dilute(x_img, s)
    merged = lax.dynamic_update_slice(tc_out, sc_out, (0, 0, 0))
    return merged


def kernel(x):
    return _dilute(x).reshape(BATCH, CHANNELS, TO_H, TO_W)


# speedup vs baseline: 1.0006x; 1.0006x over previous
import jax
import jax.numpy as jnp
import numpy as np
from jax import lax
from jax.experimental import pallas as pl
from jax.experimental.pallas import tpu as pltpu
from jax.experimental.pallas import tpu_sc as plsc

FROM_H = FROM_W = 224
TO_H = TO_W = 512
BATCH = 4
CHANNELS = 96
IMGS = BATCH * CHANNELS

NW = 32
NIMG_PER_W = IMGS // NW
CHUNK = 4
IN_ROWS = CHUNK * 7
OUT_ROWS = CHUNK * 16
INW = IN_ROWS * FROM_W
NCHUNK16 = INW // 16
NBUF = 2
HALF_ROWS = FROM_H // 2
HALF_OUT = TO_H // 2

P7 = tuple(int(v) for v in (np.arange(7) * 16) // 7)


def _build_idx_table() -> np.ndarray:
    xmap = (np.arange(FROM_W) * TO_W) // FROM_W
    idx = np.empty((CHUNK, 7, FROM_W), np.int32)
    for rg in range(CHUNK):
        for j in range(7):
            idx[rg, j, :] = ((rg * 16 + P7[j]) << 9) | xmap
    return idx.reshape(-1)


_IDX_TABLE = _build_idx_table()


def _sc_body(x_hbm, idx_hbm, out_hbm, idxv,
             in0, in1, out0, out1, isem0, isem1, osem0, osem1):
    inbufs = (in0, in1)
    outbufs = (out0, out1)
    isems = (isem0, isem1)
    osems = (osem0, osem1)

    wid = lax.axis_index("s") * 2 + lax.axis_index("c")
    img_base = wid * NIMG_PER_W

    pltpu.sync_copy(idx_hbm, idxv)

    zv = jnp.zeros((16,), jnp.float32)

    @pl.loop(0, OUT_ROWS)
    def _zero_r(r):
        for b in range(NBUF):
            @pl.loop(0, TO_W // 16, unroll=8)
            def _zero_c(c):
                outbufs[b][r, pl.ds(c * 16, 16)] = zv

    def in_slice(img, half):
        return x_hbm.at[img, pl.ds(half * HALF_ROWS, HALF_ROWS)]

    for half in range(2):
        pltpu.async_copy(in_slice(img_base, half), inbufs[half], isems[half])

    @pl.loop(0, NIMG_PER_W)
    def _image(li):
        img = img_base + li
        for half in range(2):
            pltpu.make_async_copy(
                in_slice(img, half), inbufs[half], isems[half]).wait()
            inb = inbufs[half]
            for sub in range(4):
                b = sub % 2
                outb = outbufs[b]
                r0 = half * HALF_OUT + sub * OUT_ROWS
                out_ref = out_hbm.at[img, pl.ds(r0, OUT_ROWS)]

                if half == 0 and sub < 2:
                    @pl.when(li > 0)
                    def _wait_out():
                        pltpu.make_async_copy(outb, out_ref, osems[b]).wait()
                else:
                    pltpu.make_async_copy(outb, out_ref, osems[b]).wait()

                row0 = sub * IN_ROWS

                @plsc.parallel_loop(0, NCHUNK16, 1, unroll=14)
                def _scatter(ci):
                    r = ci // (FROM_W // 16)
                    c = ci % (FROM_W // 16)
                    iv = idxv[pl.ds(ci * 16, 16)]
                    ivr = lax.shift_right_logical(iv, 9)
                    ivc = lax.bitwise_and(iv, 511)
                    dv = inb[row0 + r, pl.ds(c * 16, 16)]
                    plsc.store_scatter(outb, [ivr, ivc], dv)

                pltpu.async_copy(outb, out_ref, osems[b])

            @pl.when(li + 1 < NIMG_PER_W)
            def _next_in():
                pltpu.async_copy(
                    in_slice(img + 1, half), inbufs[half], isems[half])

    for b in range(2):
        pltpu.make_async_copy(
            outbufs[b], out_hbm.at[img_base, pl.ds(0, OUT_ROWS)],
            osems[b]).wait()


@jax.jit
def _dilute(x):
    x_img = x.reshape(IMGS, FROM_H, FROM_W)
    idx = jnp.asarray(_IDX_TABLE)
    call = pl.kernel(
        _sc_body,
        out_type=jax.ShapeDtypeStruct((IMGS, TO_H, TO_W), jnp.float32),
        mesh=plsc.VectorSubcoreMesh(core_axis_name="c", subcore_axis_name="s"),
        compiler_params=pltpu.CompilerParams(needs_layout_passes=False),
        scratch_types=(
            [pltpu.VMEM((INW,), jnp.int32)]
            + [pltpu.VMEM((HALF_ROWS, FROM_W), jnp.float32) for _ in range(2)]
            + [pltpu.VMEM((OUT_ROWS, TO_W), jnp.float32) for _ in range(2)]
            + [pltpu.SemaphoreType.DMA for _ in range(4)]
        ),
    )
    return call(x_img, idx)


def kernel(x):
    return _dilute(x).reshape(BATCH, CHANNELS, TO_H, TO_W)
